# baseline (speedup 1.0000x reference)
import functools

import jax
import jax.numpy as jnp
from jax import lax
from jax.experimental import pallas as pl
from jax.experimental.pallas import tpu as pltpu
from jax.experimental.pallas import tpu_sc as plsc

B, C, OUT = 1024, 256, 256
K = 16
EPS = 1e-12
BLK = 512
FBLK = 256
NC, NS = 2, 16
NW = NC * NS
RPW = B // NW
NCHUNK = C // 16


def _sort16(v, descending):
    return plsc.sort_key_val(v, v, descending=descending)[0]


def _row_stats(row_ref, r):
    chunks = [row_ref[r, pl.ds(16 * i, 16)] for i in range(NCHUNK)]
    sq = chunks[0] * chunks[0]
    for v in chunks[1:]:
        sq = sq + v * v
    tops, bots = [], []
    for j in range(NCHUNK // 2):
        a = _sort16(chunks[2 * j], False)
        b = _sort16(chunks[2 * j + 1], True)
        tops.append(jnp.maximum(a, b))
        bots.append(jnp.minimum(a, b))
    while len(tops) > 1:
        nt, nb = [], []
        for j in range(len(tops) // 2):
            ta = _sort16(tops[2 * j], False)
            tb = _sort16(tops[2 * j + 1], True)
            nt.append(jnp.maximum(ta, tb))
            ba = _sort16(bots[2 * j], True)
            bb = _sort16(bots[2 * j + 1], False)
            nb.append(jnp.minimum(ba, bb))
        tops, bots = nt, nb
    return jnp.sum(tops[0]), jnp.sum(bots[0]), jnp.sum(sq)


@functools.partial(
    pl.kernel,
    out_type=jax.ShapeDtypeStruct((B, 16), jnp.float32),
    mesh=plsc.VectorSubcoreMesh(
        core_axis_name="c", subcore_axis_name="s",
        num_cores=NC, num_subcores=NS),
    scratch_types=[
        pltpu.VMEM((RPW, C), jnp.float32),
        pltpu.VMEM((RPW, 16), jnp.float32),
    ],
    compiler_params=pltpu.CompilerParams(
        needs_layout_passes=False, use_tc_tiling_on_sc=True),
)
def _sc_stats(fh_hbm, out_hbm, rows_v, stats_v):
    c = lax.axis_index("c")
    s = lax.axis_index("s")
    wid = s * NC + c
    base = wid * RPW
    pltpu.sync_copy(fh_hbm.at[pl.ds(base, RPW)], rows_v)
    lane = lax.iota(jnp.int32, 16)

    @plsc.parallel_loop(0, RPW, 1, unroll=4)
    def body(r):
        topsum, botsum, sqsum = _row_stats(rows_v, r)
        outv = jnp.where(
            lane == 0, topsum,
            jnp.where(lane == 1, botsum, jnp.where(lane == 2, sqsum, 0.0)))
        stats_v[r, :] = outv
    pltpu.sync_copy(stats_v, out_hbm.at[pl.ds(base, RPW)])


def _tc_mm_body(f_rad, w, a_ref, bm_ref):
    fr = f_rad[...]
    sq_r = jnp.sum(fr * fr, axis=1, keepdims=True)
    nr = fr / jnp.maximum(jnp.sqrt(sq_r), EPS)
    p = jnp.maximum(nr, 0.0)
    m = jnp.minimum(nr, 0.0)
    x1 = jnp.concatenate([p, -m], axis=1)
    x2 = jnp.concatenate([m, -p], axis=1)
    wv = w[...]
    dims = (((1,), (1,)), ((), ()))
    a_ref[...] = lax.dot_general(
        x1, wv, dims, preferred_element_type=jnp.float32,
        precision=lax.Precision.DEFAULT)
    bm_ref[...] = lax.dot_general(
        x2, wv, dims, preferred_element_type=jnp.float32,
        precision=lax.Precision.DEFAULT)


def _tc_fin_body(a, bm, stats, radf, histof, b, token, out_ref):
    st = stats[...]
    denom = jnp.float32(K) * jnp.maximum(jnp.sqrt(st[:, 2:3]), EPS)
    tp = st[:, 0:1] / denom
    bt = st[:, 1:2] / denom
    flag = 1.0 - (radf[...] * histof[...]).reshape(-1, 1)
    out_ref[...] = tp * a[...] + bt * bm[...] + b[...].reshape(1, OUT) + token[...] * flag


@jax.jit
def kernel(f_rad, f_histo, rad_mask, histo_mask, W, b, token):
    stats = _sc_stats(f_histo)
    radf = rad_mask.astype(jnp.float32)
    histof = histo_mask.astype(jnp.float32)
    a, bm = pl.pallas_call(
        _tc_mm_body,
        grid=(B // BLK,),
        in_specs=[
            pl.BlockSpec((BLK, C), lambda i: (i, 0)),
            pl.BlockSpec((OUT, 2 * C), lambda i: (0, 0)),
        ],
        out_specs=[
            pl.BlockSpec((BLK, OUT), lambda i: (i, 0)),
            pl.BlockSpec((BLK, OUT), lambda i: (i, 0)),
        ],
        out_shape=[
            jax.ShapeDtypeStruct((B, OUT), jnp.float32),
            jax.ShapeDtypeStruct((B, OUT), jnp.float32),
        ],
    )(f_rad, W)
    return pl.pallas_call(
        _tc_fin_body,
        grid=(B // FBLK,),
        in_specs=[
            pl.BlockSpec((FBLK, OUT), lambda i: (i, 0)),
            pl.BlockSpec((FBLK, OUT), lambda i: (i, 0)),
            pl.BlockSpec((FBLK, 16), lambda i: (i, 0)),
            pl.BlockSpec((FBLK,), lambda i: (i,)),
            pl.BlockSpec((FBLK,), lambda i: (i,)),
            pl.BlockSpec((OUT,), lambda i: (0,)),
            pl.BlockSpec((1, OUT), lambda i: (0, 0)),
        ],
        out_specs=pl.BlockSpec((FBLK, OUT), lambda i: (i, 0)),
        out_shape=jax.ShapeDtypeStruct((B, OUT), jnp.float32),
    )(a, bm, stats, radf, histof, b, token)


# speedup vs baseline: 1.0397x; 1.0397x over previous
import functools

import jax
import jax.numpy as jnp
from jax import lax
from jax.experimental import pallas as pl
from jax.experimental.pallas import tpu as pltpu
from jax.experimental.pallas import tpu_sc as plsc

B, C, OUT = 1024, 256, 256
K = 16
EPS = 1e-12
BLK = 512
FBLK = 512
NC, NS = 2, 16
NW = NC * NS
RPW = B // NW
NCHUNK = C // 16


def _sort16(v, descending):
    return plsc.sort_key_val(v, v, descending=descending)[0]


def _row_stats(row_ref, r):
    chunks = [row_ref[r, pl.ds(16 * i, 16)] for i in range(NCHUNK)]
    sq = chunks[0] * chunks[0]
    for v in chunks[1:]:
        sq = sq + v * v
    tops, bots = [], []
    for j in range(NCHUNK // 2):
        a = _sort16(chunks[2 * j], False)
        b = _sort16(chunks[2 * j + 1], True)
        tops.append(jnp.maximum(a, b))
        bots.append(jnp.minimum(a, b))
    while len(tops) > 1:
        nt, nb = [], []
        for j in range(len(tops) // 2):
            ta = _sort16(tops[2 * j], False)
            tb = _sort16(tops[2 * j + 1], True)
            nt.append(jnp.maximum(ta, tb))
            ba = _sort16(bots[2 * j], True)
            bb = _sort16(bots[2 * j + 1], False)
            nb.append(jnp.minimum(ba, bb))
        tops, bots = nt, nb
    return jnp.sum(tops[0]), jnp.sum(bots[0]), jnp.sum(sq)


@functools.partial(
    pl.kernel,
    out_type=jax.ShapeDtypeStruct((B, 16), jnp.float32),
    mesh=plsc.VectorSubcoreMesh(
        core_axis_name="c", subcore_axis_name="s",
        num_cores=NC, num_subcores=NS),
    scratch_types=[
        pltpu.VMEM((RPW, C), jnp.float32),
        pltpu.VMEM((RPW, 16), jnp.float32),
    ],
    compiler_params=pltpu.CompilerParams(
        needs_layout_passes=False, use_tc_tiling_on_sc=True),
)
def _sc_stats(fh_hbm, out_hbm, rows_v, stats_v):
    c = lax.axis_index("c")
    s = lax.axis_index("s")
    wid = s * NC + c
    base = wid * RPW
    pltpu.sync_copy(fh_hbm.at[pl.ds(base, RPW)], rows_v)
    lane = lax.iota(jnp.int32, 16)

    @plsc.parallel_loop(0, RPW, 1, unroll=4)
    def body(r):
        topsum, botsum, sqsum = _row_stats(rows_v, r)
        outv = jnp.where(
            lane == 0, topsum,
            jnp.where(lane == 1, botsum, jnp.where(lane == 2, sqsum, 0.0)))
        stats_v[r, :] = outv
    pltpu.sync_copy(stats_v, out_hbm.at[pl.ds(base, RPW)])


def _tc_mm_body(f_rad, w, a_ref, bm_ref):
    fr = f_rad[...]
    sq_r = jnp.sum(fr * fr, axis=1, keepdims=True)
    nr = fr / jnp.maximum(jnp.sqrt(sq_r), EPS)
    p = jnp.maximum(nr, 0.0)
    m = jnp.minimum(nr, 0.0)
    x1 = jnp.concatenate([p, -m], axis=1)
    x2 = jnp.concatenate([m, -p], axis=1)
    wv = w[...]
    dims = (((1,), (1,)), ((), ()))
    a_ref[...] = lax.dot_general(
        x1, wv, dims, preferred_element_type=jnp.float32,
        precision=lax.Precision.DEFAULT)
    bm_ref[...] = lax.dot_general(
        x2, wv, dims, preferred_element_type=jnp.float32,
        precision=lax.Precision.DEFAULT)


def _tc_fin_body(a, bm, stats, radf, histof, b, token, out_ref):
    st = stats[...]
    denom = jnp.float32(K) * jnp.maximum(jnp.sqrt(st[:, 2:3]), EPS)
    tp = st[:, 0:1] / denom
    bt = st[:, 1:2] / denom
    flag = 1.0 - (radf[...] * histof[...]).reshape(-1, 1)
    out_ref[...] = tp * a[...] + bt * bm[...] + b[...].reshape(1, OUT) + token[...] * flag


@jax.jit
def kernel(f_rad, f_histo, rad_mask, histo_mask, W, b, token):
    stats = _sc_stats(f_histo)
    radf = rad_mask.astype(jnp.float32)
    histof = histo_mask.astype(jnp.float32)
    a, bm = pl.pallas_call(
        _tc_mm_body,
        grid=(B // BLK,),
        in_specs=[
            pl.BlockSpec((BLK, C), lambda i: (i, 0)),
            pl.BlockSpec((OUT, 2 * C), lambda i: (0, 0)),
        ],
        out_specs=[
            pl.BlockSpec((BLK, OUT), lambda i: (i, 0)),
            pl.BlockSpec((BLK, OUT), lambda i: (i, 0)),
        ],
        out_shape=[
            jax.ShapeDtypeStruct((B, OUT), jnp.float32),
            jax.ShapeDtypeStruct((B, OUT), jnp.float32),
        ],
    )(f_rad, W)
    return pl.pallas_call(
        _tc_fin_body,
        grid=(B // FBLK,),
        in_specs=[
            pl.BlockSpec((FBLK, OUT), lambda i: (i, 0)),
            pl.BlockSpec((FBLK, OUT), lambda i: (i, 0)),
            pl.BlockSpec((FBLK, 16), lambda i: (i, 0)),
            pl.BlockSpec((FBLK,), lambda i: (i,)),
            pl.BlockSpec((FBLK,), lambda i: (i,)),
            pl.BlockSpec((OUT,), lambda i: (0,)),
            pl.BlockSpec((1, OUT), lambda i: (0, 0)),
        ],
        out_specs=pl.BlockSpec((FBLK, OUT), lambda i: (i, 0)),
        out_shape=jax.ShapeDtypeStruct((B, OUT), jnp.float32),
    )(a, bm, stats, radf, histof, b, token)


# speedup vs baseline: 1.0695x; 1.0287x over previous
import functools

import jax
import jax.numpy as jnp
from jax import lax
from jax.experimental import pallas as pl
from jax.experimental.pallas import tpu as pltpu
from jax.experimental.pallas import tpu_sc as plsc

B, C, OUT = 1024, 256, 256
K = 16
EPS = 1e-12
BLK = 512
FBLK = 512
NC, NS = 2, 16
NW = NC * NS
RPW = B // NW
NCHUNK = C // 16


def _sort16(v, descending):
    return plsc.sort_key_val(v, v, descending=descending)[0]


def _row_stats(row_ref, r):
    chunks = [row_ref[r, pl.ds(16 * i, 16)] for i in range(NCHUNK)]
    sq = chunks[0] * chunks[0]
    for v in chunks[1:]:
        sq = sq + v * v
    tops, bots = [], []
    for j in range(NCHUNK // 2):
        a = _sort16(chunks[2 * j], False)
        b = _sort16(chunks[2 * j + 1], True)
        tops.append(jnp.maximum(a, b))
        bots.append(jnp.minimum(a, b))
    while len(tops) > 1:
        nt, nb = [], []
        for j in range(len(tops) // 2):
            ta = _sort16(tops[2 * j], False)
            tb = _sort16(tops[2 * j + 1], True)
            nt.append(jnp.maximum(ta, tb))
            ba = _sort16(bots[2 * j], True)
            bb = _sort16(bots[2 * j + 1], False)
            nb.append(jnp.minimum(ba, bb))
        tops, bots = nt, nb
    return jnp.sum(tops[0]), jnp.sum(bots[0]), jnp.sum(sq)


@functools.partial(
    pl.kernel,
    out_type=jax.ShapeDtypeStruct((B, 16), jnp.float32),
    mesh=plsc.VectorSubcoreMesh(
        core_axis_name="c", subcore_axis_name="s",
        num_cores=NC, num_subcores=NS),
    scratch_types=[
        pltpu.VMEM((RPW, C), jnp.float32),
        pltpu.VMEM((RPW, 16), jnp.float32),
    ],
    compiler_params=pltpu.CompilerParams(
        needs_layout_passes=False, use_tc_tiling_on_sc=True),
)
def _sc_stats(fh_hbm, out_hbm, rows_v, stats_v):
    c = lax.axis_index("c")
    s = lax.axis_index("s")
    wid = s * NC + c
    base = wid * RPW
    pltpu.sync_copy(fh_hbm.at[pl.ds(base, RPW)], rows_v)
    lane = lax.iota(jnp.int32, 16)

    @plsc.parallel_loop(0, RPW, 1, unroll=2)
    def body(r):
        topsum, botsum, sqsum = _row_stats(rows_v, r)
        outv = jnp.where(
            lane == 0, topsum,
            jnp.where(lane == 1, botsum, jnp.where(lane == 2, sqsum, 0.0)))
        stats_v[r, :] = outv
    pltpu.sync_copy(stats_v, out_hbm.at[pl.ds(base, RPW)])


def _tc_mm_body(f_rad, w, a_ref, bm_ref):
    fr = f_rad[...]
    sq_r = jnp.sum(fr * fr, axis=1, keepdims=True)
    nr = fr / jnp.maximum(jnp.sqrt(sq_r), EPS)
    p = jnp.maximum(nr, 0.0)
    m = jnp.minimum(nr, 0.0)
    x1 = jnp.concatenate([p, -m], axis=1)
    x2 = jnp.concatenate([m, -p], axis=1)
    wv = w[...]
    dims = (((1,), (1,)), ((), ()))
    a_ref[...] = lax.dot_general(
        x1, wv, dims, preferred_element_type=jnp.float32,
        precision=lax.Precision.DEFAULT).astype(jnp.bfloat16)
    bm_ref[...] = lax.dot_general(
        x2, wv, dims, preferred_element_type=jnp.float32,
        precision=lax.Precision.DEFAULT).astype(jnp.bfloat16)


def _tc_fin_body(a, bm, stats, radf, histof, b, token, out_ref):
    st = stats[...]
    denom = jnp.float32(K) * jnp.maximum(jnp.sqrt(st[:, 2:3]), EPS)
    tp = st[:, 0:1] / denom
    bt = st[:, 1:2] / denom
    flag = 1.0 - (radf[...] * histof[...]).reshape(-1, 1)
    out_ref[...] = (tp * a[...].astype(jnp.float32)
                    + bt * bm[...].astype(jnp.float32)
                    + b[...].reshape(1, OUT) + token[...] * flag)


@jax.jit
def kernel(f_rad, f_histo, rad_mask, histo_mask, W, b, token):
    stats = _sc_stats(f_histo)
    radf = rad_mask.astype(jnp.float32)
    histof = histo_mask.astype(jnp.float32)
    a, bm = pl.pallas_call(
        _tc_mm_body,
        grid=(B // BLK,),
        in_specs=[
            pl.BlockSpec((BLK, C), lambda i: (i, 0)),
            pl.BlockSpec((OUT, 2 * C), lambda i: (0, 0)),
        ],
        out_specs=[
            pl.BlockSpec((BLK, OUT), lambda i: (i, 0)),
            pl.BlockSpec((BLK, OUT), lambda i: (i, 0)),
        ],
        out_shape=[
            jax.ShapeDtypeStruct((B, OUT), jnp.bfloat16),
            jax.ShapeDtypeStruct((B, OUT), jnp.bfloat16),
        ],
    )(f_rad, W)
    return pl.pallas_call(
        _tc_fin_body,
        grid=(B // FBLK,),
        in_specs=[
            pl.BlockSpec((FBLK, OUT), lambda i: (i, 0)),
            pl.BlockSpec((FBLK, OUT), lambda i: (i, 0)),
            pl.BlockSpec((FBLK, 16), lambda i: (i, 0)),
            pl.BlockSpec((FBLK,), lambda i: (i,)),
            pl.BlockSpec((FBLK,), lambda i: (i,)),
            pl.BlockSpec((OUT,), lambda i: (0,)),
            pl.BlockSpec((1, OUT), lambda i: (0, 0)),
        ],
        out_specs=pl.BlockSpec((FBLK, OUT), lambda i: (i, 0)),
        out_shape=jax.ShapeDtypeStruct((B, OUT), jnp.float32),
    )(a, bm, stats, radf, histof, b, token)
